# baseline (speedup 1.0000x reference)
import jax
import jax.numpy as jnp
from jax import lax
from jax.experimental import pallas as pl
from jax.experimental.pallas import tpu as pltpu
from jax.experimental.pallas import tpu_sc as plsc

_D = 1024
_B, _T = 4, 2048
_N = _B * _T
_NC, _NS = 2, 16
_NW = _NC * _NS
_TPW = _T // _NW
_C = 16
_NQ = _TPW // _C
_NCHUNK = _NQ * _B
_NPAIR = _NCHUNK // 2
_SL = _D // 16
_NG = 5


def _body(idx_hbm, emb_hbm, pos_hbm, out_hbm,
          idx_v, p0, p1, g0, g1, g2, g3, g4,
          sp0, sp1, sg0, sg1, sg2, sg3, sg4, so0, so1, so2, so3, so4):
    wid = lax.axis_index("s") * _NC + lax.axis_index("c")
    t0 = wid * _TPW

    pltpu.sync_copy(idx_hbm.at[wid], idx_v)

    ps = (p0, p1)
    sps = (sp0, sp1)
    gs = (g0, g1, g2, g3, g4)
    sgs = (sg0, sg1, sg2, sg3, sg4)
    sos = (so0, so1, so2, so3, so4)
    cp_out = [None] * _NCHUNK
    cp_pos = [None] * _NQ
    cp_g = [None] * _NCHUNK

    pltpu.sync_copy(pos_hbm.at[pl.ds(t0, _C)], p0)
    cp_pos[1] = pltpu.async_copy(pos_hbm.at[pl.ds(t0 + _C, _C)], p1, sp1)

    def start(k):
        slot = k % _NG
        if k >= _NG:
            cp_out[k - _NG].wait()
        cp_g[k] = pltpu.async_copy(emb_hbm.at[idx_v.at[k]], gs[slot],
                                   sgs[slot])

    for k in range(_NG - 1):
        start(k)

    for q in range(_NQ):
        if 4 * q + _NG - 1 < _NCHUNK:
            start(4 * q + _NG - 1)
        if q > 0:
            cp_pos[q].wait()
        for b in range(_B):
            cp_g[4 * q + b].wait()
        pb = ps[q % 2]
        gq = tuple(gs[(4 * q + b) % _NG] for b in range(_B))

        @plsc.parallel_loop(0, _C, 1, unroll=1)
        def add_row(r):
            for c in range(_SL):
                sl = pl.ds(c * 16, 16)
                v = pb[r, sl]
                for g in gq:
                    g[r, sl] = g[r, sl] + v

        for b in range(_B):
            k = 4 * q + b
            dst = out_hbm.at[pl.ds(b * _T + t0 + q * _C, _C)]
            cp_out[k] = pltpu.async_copy(gs[k % _NG], dst, sos[k % _NG])
        if q + 2 < _NQ:
            cp_pos[q + 2] = pltpu.async_copy(
                pos_hbm.at[pl.ds(t0 + (q + 2) * _C, _C)], pb, sps[q % 2])
        for k in range(4 * q + _NG, min(4 * q + _NG + 3, _NCHUNK)):
            start(k)

    for k in range(_NCHUNK - _NG, _NCHUNK):
        cp_out[k].wait()


@jax.jit
def _run(idx, embedding, position_embedding):
    mesh = plsc.VectorSubcoreMesh(
        core_axis_name="c", subcore_axis_name="s", num_cores=_NC,
        num_subcores=_NS)
    out = pl.kernel(
        _body,
        out_type=jax.ShapeDtypeStruct((_N, _D), jnp.float32),
        mesh=mesh,
        scratch_types=[
            pltpu.VMEM((_NCHUNK, _C), jnp.int32),
            pltpu.VMEM((_C, _D), jnp.float32),
            pltpu.VMEM((_C, _D), jnp.float32),
            pltpu.VMEM((_C, _D), jnp.float32),
            pltpu.VMEM((_C, _D), jnp.float32),
            pltpu.VMEM((_C, _D), jnp.float32),
            pltpu.VMEM((_C, _D), jnp.float32),
            pltpu.VMEM((_C, _D), jnp.float32),
            pltpu.SemaphoreType.DMA,
            pltpu.SemaphoreType.DMA,
            pltpu.SemaphoreType.DMA,
            pltpu.SemaphoreType.DMA,
            pltpu.SemaphoreType.DMA,
            pltpu.SemaphoreType.DMA,
            pltpu.SemaphoreType.DMA,
            pltpu.SemaphoreType.DMA,
            pltpu.SemaphoreType.DMA,
            pltpu.SemaphoreType.DMA,
            pltpu.SemaphoreType.DMA,
            pltpu.SemaphoreType.DMA,
        ],
        name="emb_lookup_sc",
    )(idx, embedding, position_embedding)
    return out.reshape(_B, _T, _D)


def kernel(x, embedding, position_embedding):
    idx = (x.astype(jnp.int32)
           .reshape(_B, _NW, _NQ, _C)
           .transpose(1, 2, 0, 3)
           .reshape(_NW, _NCHUNK, _C))
    return _run(idx, embedding, position_embedding)


# speedup vs baseline: 1.1744x; 1.1744x over previous
import jax
import jax.numpy as jnp
from jax import lax
from jax.experimental import pallas as pl
from jax.experimental.pallas import tpu as pltpu
from jax.experimental.pallas import tpu_sc as plsc

_D = 1024
_B, _T = 4, 2048
_N = _B * _T
_NC, _NS = 2, 16
_NW = _NC * _NS
_TPW = _T // _NW
_C = 16
_NQ = _TPW // _C
_NCHUNK = _NQ * _B
_NPAIR = _NCHUNK // 2
_SL = _D // 16
_NG = 5


def _body(idx_hbm, emb_hbm, pos_hbm, out_hbm,
          idx_v, p0, p1, g0, g1, g2, g3, g4,
          sp0, sp1, sg0, sg1, sg2, sg3, sg4, so0, so1, so2, so3, so4):
    wid = lax.axis_index("s") * _NC + lax.axis_index("c")
    t0 = wid * _TPW

    pltpu.sync_copy(idx_hbm.at[wid], idx_v)

    ps = (p0, p1)
    sps = (sp0, sp1)
    gs = (g0, g1, g2, g3, g4)
    sgs = (sg0, sg1, sg2, sg3, sg4)
    sos = (so0, so1, so2, so3, so4)
    cp_out = [None] * _NCHUNK
    cp_pos = [None] * _NQ
    cp_g = [None] * _NCHUNK

    pltpu.sync_copy(pos_hbm.at[pl.ds(t0, _C)], p0)
    cp_pos[1] = pltpu.async_copy(pos_hbm.at[pl.ds(t0 + _C, _C)], p1, sp1)

    def start(k):
        slot = k % _NG
        if k >= _NG:
            cp_out[k - _NG].wait()
        cp_g[k] = pltpu.async_copy(emb_hbm.at[idx_v.at[k]], gs[slot],
                                   sgs[slot])

    for k in range(_NG):
        start(k)

    for p in range(_NPAIR):
        ka, kb = 2 * p, 2 * p + 1
        q = p // 2
        if p % 2 == 0 and q > 0:
            cp_pos[q].wait()
        cp_g[ka].wait()
        cp_g[kb].wait()
        pb = ps[q % 2]
        ga, gb = gs[ka % _NG], gs[kb % _NG]

        @plsc.parallel_loop(0, _C, 1, unroll=1)
        def add_row(r):
            for c in range(_SL):
                sl = pl.ds(c * 16, 16)
                v = pb[r, sl]
                ga[r, sl] = ga[r, sl] + v
                gb[r, sl] = gb[r, sl] + v

        for k in (ka, kb):
            b = k % _B
            dst = out_hbm.at[pl.ds(b * _T + t0 + q * _C, _C)]
            cp_out[k] = pltpu.async_copy(gs[k % _NG], dst, sos[k % _NG])
        if p % 2 == 1 and q + 2 < _NQ:
            cp_pos[q + 2] = pltpu.async_copy(
                pos_hbm.at[pl.ds(t0 + (q + 2) * _C, _C)], pb, sps[q % 2])
        for k in (2 * p + _NG, 2 * p + _NG + 1):
            if k < _NCHUNK:
                start(k)

    for k in range(_NCHUNK - _NG, _NCHUNK):
        cp_out[k].wait()


@jax.jit
def _run(idx, embedding, position_embedding):
    mesh = plsc.VectorSubcoreMesh(
        core_axis_name="c", subcore_axis_name="s", num_cores=_NC,
        num_subcores=_NS)
    out = pl.kernel(
        _body,
        out_type=jax.ShapeDtypeStruct((_N, _D), jnp.float32),
        mesh=mesh,
        scratch_types=[
            pltpu.VMEM((_NCHUNK, _C), jnp.int32),
            pltpu.VMEM((_C, _D), jnp.float32),
            pltpu.VMEM((_C, _D), jnp.float32),
            pltpu.VMEM((_C, _D), jnp.float32),
            pltpu.VMEM((_C, _D), jnp.float32),
            pltpu.VMEM((_C, _D), jnp.float32),
            pltpu.VMEM((_C, _D), jnp.float32),
            pltpu.VMEM((_C, _D), jnp.float32),
            pltpu.SemaphoreType.DMA,
            pltpu.SemaphoreType.DMA,
            pltpu.SemaphoreType.DMA,
            pltpu.SemaphoreType.DMA,
            pltpu.SemaphoreType.DMA,
            pltpu.SemaphoreType.DMA,
            pltpu.SemaphoreType.DMA,
            pltpu.SemaphoreType.DMA,
            pltpu.SemaphoreType.DMA,
            pltpu.SemaphoreType.DMA,
            pltpu.SemaphoreType.DMA,
            pltpu.SemaphoreType.DMA,
        ],
        name="emb_lookup_sc",
    )(idx, embedding, position_embedding)
    return out.reshape(_B, _T, _D)


def kernel(x, embedding, position_embedding):
    idx = (x.astype(jnp.int32)
           .reshape(_B, _NW, _NQ, _C)
           .transpose(1, 2, 0, 3)
           .reshape(_NW, _NCHUNK, _C))
    return _run(idx, embedding, position_embedding)


# speedup vs baseline: 1.1769x; 1.0021x over previous
import jax
import jax.numpy as jnp
from jax import lax
from jax.experimental import pallas as pl
from jax.experimental.pallas import tpu as pltpu
from jax.experimental.pallas import tpu_sc as plsc

_D = 1024
_B, _T = 4, 2048
_N = _B * _T
_NC, _NS = 2, 16
_NW = _NC * _NS
_TPW = _T // _NW
_C = 16
_NQ = _TPW // _C
_NCHUNK = _NQ * _B
_NPAIR = _NCHUNK // 2
_SL = _D // 16
_NG = 5


def _body(idx_hbm, emb_hbm, pos_hbm, out_hbm,
          idx_v, p0, p1, g0, g1, g2, g3, g4,
          sp0, sp1, sg0, sg1, sg2, sg3, sg4, so0, so1, so2, so3, so4):
    wid = lax.axis_index("s") * _NC + lax.axis_index("c")
    t0 = wid * _TPW

    pltpu.sync_copy(idx_hbm.at[wid], idx_v)

    ps = (p0, p1)
    sps = (sp0, sp1)
    gs = (g0, g1, g2, g3, g4)
    sgs = (sg0, sg1, sg2, sg3, sg4)
    sos = (so0, so1, so2, so3, so4)
    cp_out = [None] * _NCHUNK
    cp_pos = [None] * _NQ
    cp_g = [None] * _NCHUNK

    def start(k):
        slot = k % _NG
        if k >= _NG:
            cp_out[k - _NG].wait()
        cp_g[k] = pltpu.async_copy(emb_hbm.at[idx_v.at[k]], gs[slot],
                                   sgs[slot])

    for k in range(_NG):
        start(k)
    pltpu.sync_copy(pos_hbm.at[pl.ds(t0, _C)], p0)
    cp_pos[1] = pltpu.async_copy(pos_hbm.at[pl.ds(t0 + _C, _C)], p1, sp1)

    for p in range(_NPAIR):
        ka, kb = 2 * p, 2 * p + 1
        q = p // 2
        if p % 2 == 0 and q > 0:
            cp_pos[q].wait()
        cp_g[ka].wait()
        cp_g[kb].wait()
        pb = ps[q % 2]
        ga, gb = gs[ka % _NG], gs[kb % _NG]

        @plsc.parallel_loop(0, _C, 1, unroll=1)
        def add_row(r):
            for c in range(_SL):
                sl = pl.ds(c * 16, 16)
                v = pb[r, sl]
                ga[r, sl] = ga[r, sl] + v
                gb[r, sl] = gb[r, sl] + v

        for k in (ka, kb):
            b = k % _B
            dst = out_hbm.at[pl.ds(b * _T + t0 + q * _C, _C)]
            cp_out[k] = pltpu.async_copy(gs[k % _NG], dst, sos[k % _NG])
        if p % 2 == 1 and q + 2 < _NQ:
            cp_pos[q + 2] = pltpu.async_copy(
                pos_hbm.at[pl.ds(t0 + (q + 2) * _C, _C)], pb, sps[q % 2])
        for k in (2 * p + _NG, 2 * p + _NG + 1):
            if k < _NCHUNK:
                start(k)

    for k in range(_NCHUNK - _NG, _NCHUNK):
        cp_out[k].wait()


@jax.jit
def _run(idx, embedding, position_embedding):
    mesh = plsc.VectorSubcoreMesh(
        core_axis_name="c", subcore_axis_name="s", num_cores=_NC,
        num_subcores=_NS)
    out = pl.kernel(
        _body,
        out_type=jax.ShapeDtypeStruct((_N, _D), jnp.float32),
        mesh=mesh,
        scratch_types=[
            pltpu.VMEM((_NCHUNK, _C), jnp.int32),
            pltpu.VMEM((_C, _D), jnp.float32),
            pltpu.VMEM((_C, _D), jnp.float32),
            pltpu.VMEM((_C, _D), jnp.float32),
            pltpu.VMEM((_C, _D), jnp.float32),
            pltpu.VMEM((_C, _D), jnp.float32),
            pltpu.VMEM((_C, _D), jnp.float32),
            pltpu.VMEM((_C, _D), jnp.float32),
            pltpu.SemaphoreType.DMA,
            pltpu.SemaphoreType.DMA,
            pltpu.SemaphoreType.DMA,
            pltpu.SemaphoreType.DMA,
            pltpu.SemaphoreType.DMA,
            pltpu.SemaphoreType.DMA,
            pltpu.SemaphoreType.DMA,
            pltpu.SemaphoreType.DMA,
            pltpu.SemaphoreType.DMA,
            pltpu.SemaphoreType.DMA,
            pltpu.SemaphoreType.DMA,
            pltpu.SemaphoreType.DMA,
        ],
        name="emb_lookup_sc",
    )(idx, embedding, position_embedding)
    return out.reshape(_B, _T, _D)


def kernel(x, embedding, position_embedding):
    idx = (x.astype(jnp.int32)
           .reshape(_B, _NW, _NQ, _C)
           .transpose(1, 2, 0, 3)
           .reshape(_NW, _NCHUNK, _C))
    return _run(idx, embedding, position_embedding)
